# baseline (speedup 1.0000x reference)
import functools

import jax
import jax.numpy as jnp
from jax import lax
from jax.experimental import pallas as pl
from jax.experimental.pallas import tpu as pltpu
from jax.experimental.pallas import tpu_sc as plsc

N_TABLE = 100000
D = 128
BATCH = 4096
SEQ = 50
SEQ_PAD = 56
NC, NS = 2, 16
NW = NC * NS
ROWS_W = BATCH // NW
R = 2
G_IDX = R * SEQ_PAD
NCHUNK = ROWS_W // R
NBUF = 4
NGROUP = NCHUNK // NBUF

_MESH = plsc.VectorSubcoreMesh(
    core_axis_name="c", subcore_axis_name="s", num_cores=NC, num_subcores=NS
)


@functools.partial(
    pl.kernel,
    out_type=jax.ShapeDtypeStruct((BATCH, SEQ, D), jnp.float32),
    mesh=_MESH,
    scratch_types=[
        pltpu.VMEM((ROWS_W * SEQ_PAD,), jnp.int32),
        pltpu.VMEM((NBUF, G_IDX, D), jnp.float32),
        pltpu.SemaphoreType.DMA((NBUF,)),
        pltpu.SemaphoreType.DMA((NBUF,)),
    ],
)
def _sc_gather(lut_hbm, idx_hbm, out_hbm, idx_v, rows_v, gsem, ssem):
    wid = lax.axis_index("s") * NC + lax.axis_index("c")
    row0 = wid * ROWS_W
    pltpu.sync_copy(idx_hbm.at[pl.ds(row0 * SEQ_PAD, ROWS_W * SEQ_PAD)], idx_v)

    def start_gather(j, b):
        idx_slice = idx_v.at[pl.ds(j * G_IDX, G_IDX)]
        pltpu.async_copy(lut_hbm.at[idx_slice], rows_v.at[b], gsem.at[b])

    def wait_gather(b):
        idx_slice = idx_v.at[pl.ds(0, G_IDX)]
        pltpu.make_async_copy(lut_hbm.at[idx_slice], rows_v.at[b], gsem.at[b]).wait()

    def start_stores(j, b):
        for r in range(R):
            pltpu.async_copy(
                rows_v.at[b, pl.ds(r * SEQ_PAD, SEQ)],
                out_hbm.at[row0 + j * R + r],
                ssem.at[b],
            )

    def wait_stores(b):
        for _ in range(R):
            pltpu.make_async_copy(
                rows_v.at[b, pl.ds(0, SEQ)], out_hbm.at[row0], ssem.at[b]
            ).wait()

    for b in range(NBUF - 1):
        start_gather(b, b)

    for b in range(NBUF):
        wait_gather(b)
        start_stores(b, b)
        if b > 0:
            wait_stores(b - 1)
        start_gather(b + NBUF - 1, (b - 1) % NBUF)

    def group_body(g, carry):
        j0 = g * NBUF
        for b in range(NBUF):
            j = j0 + b
            wait_gather(b)
            start_stores(j, b)
            bb = (b - 1) % NBUF
            wait_stores(bb)
            start_gather(j + NBUF - 1, bb)
        return carry

    lax.fori_loop(1, NGROUP - 1, group_body, 0)

    j0 = NCHUNK - NBUF
    wait_gather(0)
    start_stores(j0, 0)
    wait_stores(NBUF - 1)
    start_gather(j0 + NBUF - 1, NBUF - 1)
    for b in range(1, NBUF):
        wait_gather(b)
        start_stores(j0 + b, b)
    for b in range(NBUF):
        wait_stores(b)


def kernel(x, lut):
    npad = SEQ_PAD - SEQ
    pad = jnp.arange(BATCH * npad, dtype=jnp.int32) % N_TABLE
    xp = jnp.concatenate(
        [x.astype(jnp.int32), pad.reshape(BATCH, npad)], axis=1
    )
    idx = jnp.reshape(xp, (BATCH * SEQ_PAD,))
    return _sc_gather(lut, idx)


# speedup vs baseline: 1.7837x; 1.7837x over previous
import functools

import jax
import jax.numpy as jnp
from jax import lax
from jax.experimental import pallas as pl
from jax.experimental.pallas import tpu as pltpu
from jax.experimental.pallas import tpu_sc as plsc

N_TABLE = 100000
D = 128
BATCH = 4096
SEQ = 50
B_TOTAL = BATCH * SEQ
NC, NS = 2, 16
NW = NC * NS
PER_W = B_TOTAL // NW
CHUNK = 128
NCHUNK = PER_W // CHUNK
NBUF = 5
NGROUP = NCHUNK // NBUF

_MESH = plsc.VectorSubcoreMesh(
    core_axis_name="c", subcore_axis_name="s", num_cores=NC, num_subcores=NS
)


@functools.partial(
    pl.kernel,
    out_type=jax.ShapeDtypeStruct((B_TOTAL, D), jnp.float32),
    mesh=_MESH,
    scratch_types=[
        pltpu.VMEM((PER_W,), jnp.int32),
        pltpu.VMEM((NBUF, CHUNK, D), jnp.float32),
        pltpu.SemaphoreType.DMA((NBUF,)),
        pltpu.SemaphoreType.DMA((NBUF,)),
    ],
)
def _sc_gather(lut_hbm, idx_hbm, out_hbm, idx_v, rows_v, gsem, ssem):
    wid = lax.axis_index("s") * NC + lax.axis_index("c")
    base = wid * PER_W
    pltpu.sync_copy(idx_hbm.at[pl.ds(base, PER_W)], idx_v)

    def start_gather(j, b):
        idx_slice = idx_v.at[pl.ds(j * CHUNK, CHUNK)]
        pltpu.async_copy(lut_hbm.at[idx_slice], rows_v.at[b], gsem.at[b])

    def wait_gather(b):
        idx_slice = idx_v.at[pl.ds(0, CHUNK)]
        pltpu.make_async_copy(lut_hbm.at[idx_slice], rows_v.at[b], gsem.at[b]).wait()

    def start_store(j, b):
        pltpu.async_copy(
            rows_v.at[b], out_hbm.at[pl.ds(base + j * CHUNK, CHUNK)], ssem.at[b]
        )

    def wait_store(b):
        pltpu.make_async_copy(
            rows_v.at[b], out_hbm.at[pl.ds(base, CHUNK)], ssem.at[b]
        ).wait()

    for b in range(NBUF - 1):
        start_gather(b, b)

    for b in range(NBUF):
        wait_gather(b)
        start_store(b, b)
        if b > 0:
            wait_store(b - 1)
        start_gather(b + NBUF - 1, (b - 1) % NBUF)

    def group_body(g, carry):
        j0 = g * NBUF
        for b in range(NBUF):
            j = j0 + b
            wait_gather(b)
            start_store(j, b)
            bb = (b - 1) % NBUF
            wait_store(bb)
            start_gather(j + NBUF - 1, bb)
        return carry

    lax.fori_loop(1, NGROUP - 1, group_body, 0)

    j0 = NCHUNK - NBUF
    wait_gather(0)
    start_store(j0, 0)
    wait_store(NBUF - 1)
    start_gather(j0 + NBUF - 1, NBUF - 1)
    for b in range(1, NBUF):
        wait_gather(b)
        start_store(j0 + b, b)
    for b in range(NBUF):
        wait_store(b)


def kernel(x, lut):
    xs = jnp.transpose(x.astype(jnp.int32), (1, 0))
    idx = jnp.reshape(xs, (B_TOTAL,))
    out = _sc_gather(lut, idx)
    out = jnp.reshape(out, (SEQ, BATCH, D))
    return jnp.transpose(out, (1, 0, 2))


# speedup vs baseline: 1.7838x; 1.0001x over previous
import functools

import jax
import jax.numpy as jnp
from jax import lax
from jax.experimental import pallas as pl
from jax.experimental.pallas import tpu as pltpu
from jax.experimental.pallas import tpu_sc as plsc

N_TABLE = 100000
D = 128
BATCH = 4096
SEQ = 50
B_TOTAL = BATCH * SEQ
NC, NS = 2, 16
NW = NC * NS
PER_W = B_TOTAL // NW
CHUNK = 160
NCHUNK = PER_W // CHUNK
NBUF = 5
NGROUP = NCHUNK // NBUF

_MESH = plsc.VectorSubcoreMesh(
    core_axis_name="c", subcore_axis_name="s", num_cores=NC, num_subcores=NS
)


@functools.partial(
    pl.kernel,
    out_type=jax.ShapeDtypeStruct((B_TOTAL, D), jnp.float32),
    mesh=_MESH,
    scratch_types=[
        pltpu.VMEM((PER_W,), jnp.int32),
        pltpu.VMEM((NBUF, CHUNK, D), jnp.float32),
        pltpu.SemaphoreType.DMA((NBUF,)),
        pltpu.SemaphoreType.DMA((NBUF,)),
    ],
)
def _sc_gather(lut_hbm, idx_hbm, out_hbm, idx_v, rows_v, gsem, ssem):
    wid = lax.axis_index("s") * NC + lax.axis_index("c")
    base = wid * PER_W
    pltpu.sync_copy(idx_hbm.at[pl.ds(base, PER_W)], idx_v)

    def start_gather(j, b):
        idx_slice = idx_v.at[pl.ds(j * CHUNK, CHUNK)]
        pltpu.async_copy(lut_hbm.at[idx_slice], rows_v.at[b], gsem.at[b])

    def wait_gather(b):
        idx_slice = idx_v.at[pl.ds(0, CHUNK)]
        pltpu.make_async_copy(lut_hbm.at[idx_slice], rows_v.at[b], gsem.at[b]).wait()

    def start_store(j, b):
        pltpu.async_copy(
            rows_v.at[b], out_hbm.at[pl.ds(base + j * CHUNK, CHUNK)], ssem.at[b]
        )

    def wait_store(b):
        pltpu.make_async_copy(
            rows_v.at[b], out_hbm.at[pl.ds(base, CHUNK)], ssem.at[b]
        ).wait()

    for b in range(NBUF - 1):
        start_gather(b, b)

    for b in range(NBUF):
        wait_gather(b)
        start_store(b, b)
        if b > 0:
            wait_store(b - 1)
        start_gather(b + NBUF - 1, (b - 1) % NBUF)

    def group_body(g, carry):
        j0 = g * NBUF
        for b in range(NBUF):
            j = j0 + b
            wait_gather(b)
            start_store(j, b)
            bb = (b - 1) % NBUF
            wait_store(bb)
            start_gather(j + NBUF - 1, bb)
        return carry

    lax.fori_loop(1, NGROUP - 1, group_body, 0)

    j0 = NCHUNK - NBUF
    wait_gather(0)
    start_store(j0, 0)
    wait_store(NBUF - 1)
    start_gather(j0 + NBUF - 1, NBUF - 1)
    for b in range(1, NBUF):
        wait_gather(b)
        start_store(j0 + b, b)
    for b in range(NBUF):
        wait_store(b)


def kernel(x, lut):
    xs = jnp.transpose(x.astype(jnp.int32), (1, 0))
    idx = jnp.reshape(xs, (B_TOTAL,))
    out = _sc_gather(lut, idx)
    out = jnp.reshape(out, (SEQ, BATCH, D))
    return jnp.transpose(out, (1, 0, 2))


# speedup vs baseline: 1.7851x; 1.0007x over previous
import functools

import jax
import jax.numpy as jnp
from jax import lax
from jax.experimental import pallas as pl
from jax.experimental.pallas import tpu as pltpu
from jax.experimental.pallas import tpu_sc as plsc

N_TABLE = 100000
D = 128
BATCH = 4096
SEQ = 50
B_TOTAL = BATCH * SEQ
NC, NS = 2, 16
NW = NC * NS
PER_W = B_TOTAL // NW
CHUNK = 200
NCHUNK = PER_W // CHUNK
NBUF = 4
NGROUP = NCHUNK // NBUF

_MESH = plsc.VectorSubcoreMesh(
    core_axis_name="c", subcore_axis_name="s", num_cores=NC, num_subcores=NS
)


@functools.partial(
    pl.kernel,
    out_type=jax.ShapeDtypeStruct((B_TOTAL, D), jnp.float32),
    mesh=_MESH,
    scratch_types=[
        pltpu.VMEM((PER_W,), jnp.int32),
        pltpu.VMEM((NBUF, CHUNK, D), jnp.float32),
        pltpu.SemaphoreType.DMA((NBUF,)),
        pltpu.SemaphoreType.DMA((NBUF,)),
    ],
)
def _sc_gather(lut_hbm, idx_hbm, out_hbm, idx_v, rows_v, gsem, ssem):
    wid = lax.axis_index("s") * NC + lax.axis_index("c")
    base = wid * PER_W
    pltpu.sync_copy(idx_hbm.at[pl.ds(base, PER_W)], idx_v)

    def start_gather(j, b):
        idx_slice = idx_v.at[pl.ds(j * CHUNK, CHUNK)]
        pltpu.async_copy(lut_hbm.at[idx_slice], rows_v.at[b], gsem.at[b])

    def wait_gather(b):
        idx_slice = idx_v.at[pl.ds(0, CHUNK)]
        pltpu.make_async_copy(lut_hbm.at[idx_slice], rows_v.at[b], gsem.at[b]).wait()

    def start_store(j, b):
        pltpu.async_copy(
            rows_v.at[b], out_hbm.at[pl.ds(base + j * CHUNK, CHUNK)], ssem.at[b]
        )

    def wait_store(b):
        pltpu.make_async_copy(
            rows_v.at[b], out_hbm.at[pl.ds(base, CHUNK)], ssem.at[b]
        ).wait()

    for b in range(NBUF - 1):
        start_gather(b, b)

    for b in range(NBUF):
        wait_gather(b)
        start_store(b, b)
        if b > 0:
            wait_store(b - 1)
        start_gather(b + NBUF - 1, (b - 1) % NBUF)

    def group_body(g, carry):
        j0 = g * NBUF
        for b in range(NBUF):
            j = j0 + b
            wait_gather(b)
            start_store(j, b)
            bb = (b - 1) % NBUF
            wait_store(bb)
            start_gather(j + NBUF - 1, bb)
        return carry

    lax.fori_loop(1, NGROUP - 1, group_body, 0)

    j0 = NCHUNK - NBUF
    wait_gather(0)
    start_store(j0, 0)
    wait_store(NBUF - 1)
    start_gather(j0 + NBUF - 1, NBUF - 1)
    for b in range(1, NBUF):
        wait_gather(b)
        start_store(j0 + b, b)
    for b in range(NBUF):
        wait_store(b)


def kernel(x, lut):
    xs = jnp.transpose(x.astype(jnp.int32), (1, 0))
    idx = jnp.reshape(xs, (B_TOTAL,))
    out = _sc_gather(lut, idx)
    out = jnp.reshape(out, (SEQ, BATCH, D))
    return jnp.transpose(out, (1, 0, 2))
